# baseline (speedup 1.0000x reference)
import jax
import jax.numpy as jnp
from jax.experimental import pallas as pl

_MAX_ITERS = 5
_DAMPING = 0.5


def kernel(evidence_logits, t1_indices, t1_weights, t2_indices, t2_weights, t3_indices, t3_weights):
    def t1(q):
        q_a = q[:, t1_indices[:, 0]]
        q_c = q[:, t1_indices[:, 1]]
        w = t1_weights[None, :]
        d = jnp.zeros_like(q)
        d = d.at[:, t1_indices[:, 1]].add(w * q_a)
        d = d.at[:, t1_indices[:, 0]].add(w * (q_c - 1.0))
        return d

    def t2(q):
        q_a = q[:, t2_indices[:, 0]]
        q_b = q[:, t2_indices[:, 1]]
        q_c = q[:, t2_indices[:, 2]]
        w = t2_weights[None, :]
        d = jnp.zeros_like(q)
        d = d.at[:, t2_indices[:, 2]].add(w * q_a * q_b)
        d = d.at[:, t2_indices[:, 0]].add(w * q_b * (q_c - 1.0))
        d = d.at[:, t2_indices[:, 1]].add(w * q_a * (q_c - 1.0))
        return d

    def t3(q):
        q_a = q[:, t3_indices[:, 0]]
        q_b = q[:, t3_indices[:, 1]]
        q_c = q[:, t3_indices[:, 2]]
        w = t3_weights[None, :]
        d = jnp.zeros_like(q)
        d = d.at[:, t3_indices[:, 2]].add(w * q_a * (1.0 - q_b))
        d = d.at[:, t3_indices[:, 0]].add(w * (1.0 - q_b) * (q_c - 1.0))
        d = d.at[:, t3_indices[:, 1]].add(w * q_a * (1.0 - q_c))
        return d

    curr_logits = evidence_logits
    curr_q = jax.nn.sigmoid(curr_logits)
    for _ in range(_MAX_ITERS):
        total = t1(curr_q) + t2(curr_q) + t3(curr_q)
        target = evidence_logits + total
        new_logits = (1.0 - _DAMPING) * curr_logits + _DAMPING * target
        curr_q = jax.nn.sigmoid(new_logits)
        curr_logits = new_logits
    return curr_q


# speedup vs baseline: 30.1341x; 30.1341x over previous
import functools

import jax
import jax.numpy as jnp
from jax import lax
from jax.experimental import pallas as pl
from jax.experimental.pallas import tpu as pltpu
from jax.experimental.pallas import tpu_sc as plsc

MAX_ITERS = 5
DAMPING = 0.5

NCORE = 2
NSUB = 16
LANES = 16
TPR = 8

KCH = 1024
CROWS = KCH // 128
U = 1792


def _pad_cols(idx, w, r_pad):
    r = w.shape[0]
    cols = [jnp.pad(idx[:, k], (0, r_pad - r)).reshape(-1, 128)
            for k in range(idx.shape[1])]
    wp = jnp.pad(w, (0, r_pad - r)).reshape(-1, 128)
    return cols, wp


def _sigmoid16(x):
    return 1.0 / (1.0 + jnp.exp(-x))


def kernel(evidence_logits, t1_indices, t1_weights, t2_indices, t2_weights,
           t3_indices, t3_weights):
    B, N = evidence_logits.shape
    R = t1_weights.shape[0]

    SPC = -(-N // (TPR * U))
    SL = SPC * U
    NPAD = TPR * SL

    CPT = -(-R // (TPR * KCH))
    RPAD = TPR * CPT * KCH
    TROWS = CPT * CROWS

    ev = jnp.pad(evidence_logits, ((0, 0), (0, NPAD - N)))
    (a1, c1), w1 = _pad_cols(t1_indices, t1_weights, RPAD)
    (a2, b2, c2), w2 = _pad_cols(t2_indices, t2_weights, RPAD)
    (a3, b3, c3), w3 = _pad_cols(t3_indices, t3_weights, RPAD)

    mesh = plsc.VectorSubcoreMesh(core_axis_name="c", subcore_axis_name="s",
                                  num_cores=NCORE, num_subcores=NSUB)

    @functools.partial(
        pl.kernel,
        out_type=(jax.ShapeDtypeStruct((B, NPAD), jnp.float32),
                  jax.ShapeDtypeStruct((B, NPAD), jnp.float32)),
        mesh=mesh,
        compiler_params=pltpu.CompilerParams(needs_layout_passes=False,
                                             use_tc_tiling_on_sc=False),
        scratch_types=[
            pltpu.VMEM((NPAD,), jnp.float32),
            pltpu.VMEM((CROWS, 128), jnp.int32),
            pltpu.VMEM((CROWS, 128), jnp.int32),
            pltpu.VMEM((CROWS, 128), jnp.int32),
            pltpu.VMEM((CROWS, 128), jnp.float32),
            pltpu.VMEM((CROWS, 128), jnp.float32),
            pltpu.VMEM((CROWS, 128), jnp.float32),
            pltpu.VMEM((CROWS, 128), jnp.float32),
            pltpu.VMEM((U,), jnp.float32),
            pltpu.VMEM((U,), jnp.float32),
            pltpu.VMEM((U,), jnp.float32),
            pltpu.VMEM_SHARED((2, NPAD), jnp.float32),
            pltpu.SemaphoreType.DMA,
            pltpu.SemaphoreType.DMA,
        ],
    )
    def cavi(ev_h, a1_h, c1_h, w1_h, a2_h, b2_h, c2_h, w2_h,
             a3_h, b3_h, c3_h, w3_h, q_h, lg_h,
             q_loc, ia, ib, ic, wb, mA, mB, mC, lg_u, dl_u, ev_u,
             dl_sh, sem_in, sem_sc):
        cid = lax.axis_index("c")
        sid = lax.axis_index("s")
        rl = sid // TPR
        b = 2 * cid + rl
        j = sid % TPR

        def update_slice(t):
            for u in range(SPC):
                off = j * SL + u * U
                pltpu.sync_copy(dl_sh.at[rl, pl.ds(off, U)], dl_u)
                pltpu.sync_copy(ev_h.at[b, pl.ds(off, U)], ev_u)
                pltpu.sync_copy(lg_h.at[b, pl.ds(off, U)], lg_u)

                def upd(v, _):
                    sl = pl.ds(v * LANES, LANES)
                    nl = ((1.0 - DAMPING) * lg_u[sl]
                          + DAMPING * (ev_u[sl] + dl_u[sl]))
                    lg_u[sl] = nl
                    dl_u[sl] = _sigmoid16(nl)
                    return _
                lax.fori_loop(0, U // LANES, upd, None)

                pltpu.sync_copy(lg_u, lg_h.at[b, pl.ds(off, U)])
                pltpu.sync_copy(dl_u, q_h.at[b, pl.ds(off, U)])

                def zero(v, _):
                    ev_u[pl.ds(v * LANES, LANES)] = jnp.zeros(
                        (LANES,), jnp.float32)
                    return _
                lax.fori_loop(0, U // LANES, zero, None)
                pltpu.sync_copy(ev_u, dl_sh.at[rl, pl.ds(off, U)])

        def run_template(idx_bufs, idx_hs, w_h, compute):
            nops = len(idx_bufs)

            def chunk(ci, _):
                row0 = j * TROWS + ci * CROWS
                cps = [pltpu.async_copy(h.at[pl.ds(row0, CROWS)], buf, sem_in)
                       for buf, h in zip(idx_bufs, idx_hs)]
                cps.append(pltpu.async_copy(
                    w_h.at[pl.ds(row0, CROWS)], wb, sem_in))
                for cp in cps:
                    cp.wait()

                def grp(r, _):
                    for l in range(128 // LANES):
                        sl = pl.ds(l * LANES, LANES)
                        ivs = [buf[r, sl] for buf in idx_bufs]
                        qs = [plsc.load_gather(q_loc, [iv]) for iv in ivs]
                        wB = wb[r, sl]
                        msgs = compute(wB, *qs)
                        for mbuf, m in zip((mA, mB, mC)[:nops], msgs):
                            mbuf[r, sl] = m
                    return _
                lax.fori_loop(0, CROWS, grp, None)

                scs = []
                for r in range(CROWS):
                    for mbuf, buf in zip((mA, mB, mC)[:nops], idx_bufs):
                        scs.append(pltpu.async_copy(
                            mbuf.at[r], dl_sh.at[rl].at[buf.at[r]],
                            sem_sc, add=True))
                for cp in scs:
                    cp.wait()
                return _

            lax.fori_loop(0, CPT, chunk, None)

        def t1_msgs(w, qa, qc):
            t = qc - 1.0
            return (w * t, w * qa)

        def t2_msgs(w, qa, qb, qc):
            t = qc - 1.0
            wqa = w * qa
            return (w * qb * t, wqa * t, wqa * qb)

        def t3_msgs(w, qa, qb, qc):
            t = qc - 1.0
            u1 = 1.0 - qb
            wqa = w * qa
            return (w * u1 * t, wqa * (1.0 - qc), wqa * u1)

        for u in range(SPC):
            off = j * SL + u * U
            pltpu.sync_copy(ev_h.at[b, pl.ds(off, U)], ev_u)
            pltpu.sync_copy(ev_u, lg_h.at[b, pl.ds(off, U)])

            def sig(v, _):
                sl = pl.ds(v * LANES, LANES)
                dl_u[sl] = _sigmoid16(ev_u[sl])
                return _
            lax.fori_loop(0, U // LANES, sig, None)
            pltpu.sync_copy(dl_u, q_h.at[b, pl.ds(off, U)])

            def zero(v, _):
                ev_u[pl.ds(v * LANES, LANES)] = jnp.zeros((LANES,), jnp.float32)
                return _
            lax.fori_loop(0, U // LANES, zero, None)
            pltpu.sync_copy(ev_u, dl_sh.at[rl, pl.ds(off, U)])

        plsc.subcore_barrier()
        pltpu.sync_copy(q_h.at[b], q_loc)

        def iteration(t, _):
            run_template([ia, ic], [a1_h, c1_h], w1_h, t1_msgs)
            run_template([ia, ib, ic], [a2_h, b2_h, c2_h], w2_h, t2_msgs)
            run_template([ia, ib, ic], [a3_h, b3_h, c3_h], w3_h, t3_msgs)
            plsc.subcore_barrier()
            update_slice(t)
            plsc.subcore_barrier()

            @pl.when(t < MAX_ITERS - 1)
            def _():
                pltpu.sync_copy(q_h.at[b], q_loc)
            return _

        lax.fori_loop(0, MAX_ITERS, iteration, None)

    out, _lg = cavi(ev, a1, c1, w1, a2, b2, c2, w2, a3, b3, c3, w3)
    return out[:, :N]


# speedup vs baseline: 59.0792x; 1.9605x over previous
import functools

import jax
import jax.numpy as jnp
from jax import lax
from jax.experimental import pallas as pl
from jax.experimental.pallas import tpu as pltpu
from jax.experimental.pallas import tpu_sc as plsc

MAX_ITERS = 5
DAMPING = 0.5

NCORE = 2
NSUB = 16
LANES = 16
TPR = 8

KCH = 640
CROWS = KCH // 128
U = 1568


def _pad_cols(idx, w, r_pad):
    r = w.shape[0]
    cols = [jnp.pad(idx[:, k], (0, r_pad - r)).reshape(-1, 128)
            for k in range(idx.shape[1])]
    wp = jnp.pad(w, (0, r_pad - r)).reshape(-1, 128)
    return cols, wp


def _sigmoid16(x):
    return 1.0 / (1.0 + jnp.exp(-x))


def kernel(evidence_logits, t1_indices, t1_weights, t2_indices, t2_weights,
           t3_indices, t3_weights):
    B, N = evidence_logits.shape
    R = t1_weights.shape[0]

    SPC = -(-N // (TPR * U))
    SL = SPC * U
    NPAD = TPR * SL

    CPT = -(-R // (TPR * KCH))
    CPT += CPT % 2
    RPAD = TPR * CPT * KCH
    TROWS = CPT * CROWS

    ev = jnp.pad(evidence_logits, ((0, 0), (0, NPAD - N)))
    (a1, c1), w1 = _pad_cols(t1_indices, t1_weights, RPAD)
    (a2, b2, c2), w2 = _pad_cols(t2_indices, t2_weights, RPAD)
    (a3, b3, c3), w3 = _pad_cols(t3_indices, t3_weights, RPAD)

    mesh = plsc.VectorSubcoreMesh(core_axis_name="c", subcore_axis_name="s",
                                  num_cores=NCORE, num_subcores=NSUB)

    cbuf_i = pltpu.VMEM((CROWS, 128), jnp.int32)
    cbuf_f = pltpu.VMEM((CROWS, 128), jnp.float32)

    @functools.partial(
        pl.kernel,
        out_type=(jax.ShapeDtypeStruct((B, NPAD), jnp.float32),
                  jax.ShapeDtypeStruct((B, NPAD), jnp.float32)),
        mesh=mesh,
        compiler_params=pltpu.CompilerParams(needs_layout_passes=False,
                                             use_tc_tiling_on_sc=False),
        scratch_types=[
            pltpu.VMEM((NPAD,), jnp.float32),
            [cbuf_i] * 3 + [cbuf_f] * 4,
            [cbuf_i] * 3 + [cbuf_f] * 4,
            [cbuf_i] * 3,
            [cbuf_i] * 3,
            pltpu.VMEM((U,), jnp.float32),
            pltpu.VMEM((U,), jnp.float32),
            pltpu.VMEM((U,), jnp.float32),
            pltpu.VMEM_SHARED((2, NPAD), jnp.float32),
            pltpu.SemaphoreType.DMA,
            pltpu.SemaphoreType.DMA,
            pltpu.SemaphoreType.DMA,
            pltpu.SemaphoreType.DMA,
        ],
    )
    def cavi(ev_h, a1_h, c1_h, w1_h, a2_h, b2_h, c2_h, w2_h,
             a3_h, b3_h, c3_h, w3_h, q_h, lg_h,
             q_loc, bufs0, bufs1, sidx0, sidx1, lg_u, dl_u, ev_u,
             dl_sh, sem_in0, sem_in1, sem_sc, sem_up):
        sem_in = (sem_in0, sem_in1)
        cid = lax.axis_index("c")
        sid = lax.axis_index("s")
        rl = sid // TPR
        b = 2 * cid + rl
        j = sid % TPR

        bufs = (bufs0, bufs1)
        sidx = (sidx0, sidx1)

        def update_slice(t):
            for u in range(SPC):
                off = j * SL + u * U
                pltpu.sync_copy(dl_sh.at[rl, pl.ds(off, U)], dl_u)
                pltpu.sync_copy(ev_h.at[b, pl.ds(off, U)], ev_u)
                pltpu.sync_copy(lg_h.at[b, pl.ds(off, U)], lg_u)

                def upd(v, _):
                    sl = pl.ds(v * LANES, LANES)
                    nl = ((1.0 - DAMPING) * lg_u[sl]
                          + DAMPING * (ev_u[sl] + dl_u[sl]))
                    lg_u[sl] = nl
                    dl_u[sl] = _sigmoid16(nl)
                    ev_u[sl] = jnp.zeros((LANES,), jnp.float32)
                    return _
                lax.fori_loop(0, U // LANES, upd, None)

                pltpu.sync_copy(lg_u, lg_h.at[b, pl.ds(off, U)])
                pltpu.sync_copy(dl_u, q_h.at[b, pl.ds(off, U)])
                pltpu.sync_copy(ev_u, dl_sh.at[rl, pl.ds(off, U)])

        def run_template(nops, idx_hs, w_h, compute):
            hs = list(idx_hs) + [w_h]

            def fire_input(ci, p):
                for h, buf in zip(hs, bufs[p][:nops] + [bufs[p][3]]):
                    row0 = j * TROWS + ci * CROWS
                    pltpu.async_copy(h.at[pl.ds(row0, CROWS)], buf, sem_in[p])

            def wait_input(ci, p):
                for h, buf in zip(hs, bufs[p][:nops] + [bufs[p][3]]):
                    row0 = j * TROWS + ci * CROWS
                    pltpu.make_async_copy(h.at[pl.ds(row0, CROWS)], buf,
                                          sem_in[p]).wait()

            def fire_scatter(p):
                for r in range(CROWS):
                    for mbuf, sbuf in zip(bufs[p][4:4 + nops], sidx[p]):
                        pltpu.async_copy(
                            mbuf.at[r], dl_sh.at[rl].at[sbuf.at[r]],
                            sem_sc, add=True)

            def wait_scatter(p):
                for mbuf in bufs[p][4:4 + nops]:
                    pltpu.make_async_copy(
                        w_h.at[pl.ds(0, CROWS)], mbuf, sem_sc).wait()

            def compute_chunk(p):
                ibs = bufs[p][:nops]
                wb = bufs[p][3]
                mbs = bufs[p][4:4 + nops]
                sbs = sidx[p]

                def grp(r, _):
                    for l in range(128 // LANES):
                        sl = pl.ds(l * LANES, LANES)
                        ivs = [buf[r, sl] for buf in ibs]
                        qs = [plsc.load_gather(q_loc, [iv]) for iv in ivs]
                        msgs = compute(wb[r, sl], *qs)
                        for mbuf, m in zip(mbs, msgs):
                            mbuf[r, sl] = m
                        for sbuf, iv in zip(sbs, ivs):
                            sbuf[r, sl] = iv
                    return _
                lax.fori_loop(0, CROWS, grp, None)

            def sub_body(ci, p, first=False):
                fire_input(lax.rem(ci + 1, CPT), 1 - p)
                wait_input(ci, p)
                compute_chunk(p)
                if not first:
                    wait_scatter(1 - p)
                fire_scatter(p)

            fire_input(0, 0)
            sub_body(0, 0, first=True)
            sub_body(1, 1)

            def pair(g, _):
                sub_body(2 * g, 0)
                sub_body(2 * g + 1, 1)
                return _
            lax.fori_loop(1, CPT // 2, pair, None)
            wait_scatter(1)
            wait_input(0, 0)

        def t1_msgs(w, qa, qc):
            t = qc - 1.0
            return (w * t, w * qa)

        def t2_msgs(w, qa, qb, qc):
            t = qc - 1.0
            wqa = w * qa
            return (w * qb * t, wqa * t, wqa * qb)

        def t3_msgs(w, qa, qb, qc):
            t = qc - 1.0
            u1 = 1.0 - qb
            wqa = w * qa
            return (w * u1 * t, wqa * (1.0 - qc), wqa * u1)

        for u in range(SPC):
            off = j * SL + u * U
            pltpu.sync_copy(ev_h.at[b, pl.ds(off, U)], ev_u)
            pltpu.sync_copy(ev_u, lg_h.at[b, pl.ds(off, U)])

            def sig(v, _):
                sl = pl.ds(v * LANES, LANES)
                dl_u[sl] = _sigmoid16(ev_u[sl])
                ev_u[sl] = jnp.zeros((LANES,), jnp.float32)
                return _
            lax.fori_loop(0, U // LANES, sig, None)
            pltpu.sync_copy(dl_u, q_h.at[b, pl.ds(off, U)])
            pltpu.sync_copy(ev_u, dl_sh.at[rl, pl.ds(off, U)])

        plsc.subcore_barrier()
        pltpu.sync_copy(q_h.at[b], q_loc)

        def iteration(t, _):
            run_template(2, [a1_h, c1_h], w1_h, t1_msgs)
            run_template(3, [a2_h, b2_h, c2_h], w2_h, t2_msgs)
            run_template(3, [a3_h, b3_h, c3_h], w3_h, t3_msgs)
            plsc.subcore_barrier()
            update_slice(t)
            plsc.subcore_barrier()

            @pl.when(t < MAX_ITERS - 1)
            def _():
                pltpu.sync_copy(q_h.at[b], q_loc)
            return _

        lax.fori_loop(0, MAX_ITERS, iteration, None)

    out, _lg = cavi(ev, a1, c1, w1, a2, b2, c2, w2, a3, b3, c3, w3)
    return out[:, :N]


# speedup vs baseline: 59.8257x; 1.0126x over previous
import functools

import jax
import jax.numpy as jnp
from jax import lax
from jax.experimental import pallas as pl
from jax.experimental.pallas import tpu as pltpu
from jax.experimental.pallas import tpu_sc as plsc

MAX_ITERS = 5
DAMPING = 0.5

NCORE = 2
NSUB = 16
LANES = 16
TPR = 8

KCH = 640
CROWS = KCH // 128
U = 1568


def _pad_cols(idx, w, r_pad):
    r = w.shape[0]
    cols = [jnp.pad(idx[:, k], (0, r_pad - r)).reshape(-1, 128)
            for k in range(idx.shape[1])]
    wp = jnp.pad(w, (0, r_pad - r)).reshape(-1, 128)
    return cols, wp


def _sigmoid16(x):
    return 1.0 / (1.0 + jnp.exp(-x))


def kernel(evidence_logits, t1_indices, t1_weights, t2_indices, t2_weights,
           t3_indices, t3_weights):
    B, N = evidence_logits.shape
    R = t1_weights.shape[0]

    SPC = -(-N // (TPR * U))
    SL = SPC * U
    NPAD = TPR * SL

    CPT = -(-R // (TPR * KCH))
    CPT += CPT % 2
    RPAD = TPR * CPT * KCH
    TROWS = CPT * CROWS

    ev = jnp.pad(evidence_logits, ((0, 0), (0, NPAD - N)))
    (a1, c1), w1 = _pad_cols(t1_indices, t1_weights, RPAD)
    (a2, b2, c2), w2 = _pad_cols(t2_indices, t2_weights, RPAD)
    (a3, b3, c3), w3 = _pad_cols(t3_indices, t3_weights, RPAD)

    mesh = plsc.VectorSubcoreMesh(core_axis_name="c", subcore_axis_name="s",
                                  num_cores=NCORE, num_subcores=NSUB)

    cbuf_i = pltpu.VMEM((CROWS, 128), jnp.int32)
    cbuf_f = pltpu.VMEM((CROWS, 128), jnp.float32)

    @functools.partial(
        pl.kernel,
        out_type=(jax.ShapeDtypeStruct((B, NPAD), jnp.float32),
                  jax.ShapeDtypeStruct((B, NPAD), jnp.float32)),
        mesh=mesh,
        compiler_params=pltpu.CompilerParams(needs_layout_passes=False,
                                             use_tc_tiling_on_sc=False),
        scratch_types=[
            pltpu.VMEM((NPAD,), jnp.float32),
            [cbuf_i] * 3 + [cbuf_f] * 4,
            [cbuf_i] * 3 + [cbuf_f] * 4,
            [cbuf_i] * 3,
            [cbuf_i] * 3,
            pltpu.VMEM((U,), jnp.float32),
            pltpu.VMEM((U,), jnp.float32),
            pltpu.VMEM((U,), jnp.float32),
            pltpu.VMEM_SHARED((2, NPAD), jnp.float32),
            pltpu.SemaphoreType.DMA,
            pltpu.SemaphoreType.DMA,
            pltpu.SemaphoreType.DMA,
            pltpu.SemaphoreType.DMA,
        ],
    )
    def cavi(ev_h, a1_h, c1_h, w1_h, a2_h, b2_h, c2_h, w2_h,
             a3_h, b3_h, c3_h, w3_h, q_h, lg_h,
             q_loc, bufs0, bufs1, sidx0, sidx1, lg_u, dl_u, ev_u,
             dl_sh, sem_in0, sem_in1, sem_sc, sem_up):
        sem_in = (sem_in0, sem_in1)
        cid = lax.axis_index("c")
        sid = lax.axis_index("s")
        rl = sid // TPR
        b = 2 * cid + rl
        j = sid % TPR

        bufs = (bufs0, bufs1)
        sidx = (sidx0, sidx1)

        def update_slice(t):
            for u in range(SPC):
                off = j * SL + u * U
                pltpu.sync_copy(dl_sh.at[rl, pl.ds(off, U)], dl_u)
                pltpu.sync_copy(ev_h.at[b, pl.ds(off, U)], ev_u)
                pltpu.sync_copy(lg_h.at[b, pl.ds(off, U)], lg_u)

                def upd(v, _):
                    sl = pl.ds(v * LANES, LANES)
                    nl = ((1.0 - DAMPING) * lg_u[sl]
                          + DAMPING * (ev_u[sl] + dl_u[sl]))
                    lg_u[sl] = nl
                    dl_u[sl] = _sigmoid16(nl)
                    ev_u[sl] = jnp.zeros((LANES,), jnp.float32)
                    return _
                lax.fori_loop(0, U // LANES, upd, None)

                pltpu.sync_copy(lg_u, lg_h.at[b, pl.ds(off, U)])
                pltpu.sync_copy(dl_u, q_h.at[b, pl.ds(off, U)])
                pltpu.sync_copy(ev_u, dl_sh.at[rl, pl.ds(off, U)])

        def run_template(nops, idx_hs, w_h, compute):
            hs = list(idx_hs) + [w_h]

            def fire_input(ci, p):
                for h, buf in zip(hs, bufs[p][:nops] + [bufs[p][3]]):
                    row0 = j * TROWS + ci * CROWS
                    pltpu.async_copy(h.at[pl.ds(row0, CROWS)], buf, sem_in[p])

            def wait_input(ci, p):
                for h, buf in zip(hs, bufs[p][:nops] + [bufs[p][3]]):
                    row0 = j * TROWS + ci * CROWS
                    pltpu.make_async_copy(h.at[pl.ds(row0, CROWS)], buf,
                                          sem_in[p]).wait()

            def fire_scatter(p):
                for r in range(CROWS):
                    for mbuf, sbuf in zip(bufs[p][4:4 + nops], sidx[p]):
                        pltpu.async_copy(
                            mbuf.at[r], dl_sh.at[rl].at[sbuf.at[r]],
                            sem_sc, add=True)

            def wait_scatter(p):
                for mbuf in bufs[p][4:4 + nops]:
                    pltpu.make_async_copy(
                        w_h.at[pl.ds(0, CROWS)], mbuf, sem_sc).wait()

            def compute_chunk(p):
                ibs = bufs[p][:nops]
                wb = bufs[p][3]
                mbs = bufs[p][4:4 + nops]
                sbs = sidx[p]

                @plsc.parallel_loop(0, CROWS, 1)
                def grp(r):
                    for l in range(128 // LANES):
                        sl = pl.ds(l * LANES, LANES)
                        ivs = [buf[r, sl] for buf in ibs]
                        qs = [plsc.load_gather(q_loc, [iv]) for iv in ivs]
                        msgs = compute(wb[r, sl], *qs)
                        for mbuf, m in zip(mbs, msgs):
                            mbuf[r, sl] = m
                        for sbuf, iv in zip(sbs, ivs):
                            sbuf[r, sl] = iv

            def sub_body(ci, p, first=False):
                fire_input(lax.rem(ci + 1, CPT), 1 - p)
                wait_input(ci, p)
                compute_chunk(p)
                if not first:
                    wait_scatter(1 - p)
                fire_scatter(p)

            fire_input(0, 0)
            sub_body(0, 0, first=True)
            sub_body(1, 1)

            def pair(g, _):
                sub_body(2 * g, 0)
                sub_body(2 * g + 1, 1)
                return _
            lax.fori_loop(1, CPT // 2, pair, None)
            wait_scatter(1)
            wait_input(0, 0)

        def t1_msgs(w, qa, qc):
            t = qc - 1.0
            return (w * t, w * qa)

        def t2_msgs(w, qa, qb, qc):
            t = qc - 1.0
            wqa = w * qa
            return (w * qb * t, wqa * t, wqa * qb)

        def t3_msgs(w, qa, qb, qc):
            t = qc - 1.0
            u1 = 1.0 - qb
            wqa = w * qa
            return (w * u1 * t, wqa * (1.0 - qc), wqa * u1)

        for u in range(SPC):
            off = j * SL + u * U
            pltpu.sync_copy(ev_h.at[b, pl.ds(off, U)], ev_u)
            pltpu.sync_copy(ev_u, lg_h.at[b, pl.ds(off, U)])

            def sig(v, _):
                sl = pl.ds(v * LANES, LANES)
                dl_u[sl] = _sigmoid16(ev_u[sl])
                ev_u[sl] = jnp.zeros((LANES,), jnp.float32)
                return _
            lax.fori_loop(0, U // LANES, sig, None)
            pltpu.sync_copy(dl_u, q_h.at[b, pl.ds(off, U)])
            pltpu.sync_copy(ev_u, dl_sh.at[rl, pl.ds(off, U)])

        plsc.subcore_barrier()
        pltpu.sync_copy(q_h.at[b], q_loc)

        def iteration(t, _):
            run_template(2, [a1_h, c1_h], w1_h, t1_msgs)
            run_template(3, [a2_h, b2_h, c2_h], w2_h, t2_msgs)
            run_template(3, [a3_h, b3_h, c3_h], w3_h, t3_msgs)
            plsc.subcore_barrier()
            update_slice(t)
            plsc.subcore_barrier()

            @pl.when(t < MAX_ITERS - 1)
            def _():
                pltpu.sync_copy(q_h.at[b], q_loc)
            return _

        lax.fori_loop(0, MAX_ITERS, iteration, None)

    out, _lg = cavi(ev, a1, c1, w1, a2, b2, c2, w2, a3, b3, c3, w3)
    return out[:, :N]


# speedup vs baseline: 60.6603x; 1.0140x over previous
import functools

import jax
import jax.numpy as jnp
from jax import lax
from jax.experimental import pallas as pl
from jax.experimental.pallas import tpu as pltpu
from jax.experimental.pallas import tpu_sc as plsc

MAX_ITERS = 5
DAMPING = 0.5

NCORE = 2
NSUB = 16
LANES = 16
TPR = 8

KCH = 640
CROWS = KCH // 128
U = 1568


def _pad_cols(idx, w, r_pad):
    r = w.shape[0]
    cols = [jnp.pad(idx[:, k], (0, r_pad - r)).reshape(-1, 128)
            for k in range(idx.shape[1])]
    wp = jnp.pad(w, (0, r_pad - r)).reshape(-1, 128)
    return cols, wp


def _sigmoid16(x):
    return 1.0 / (1.0 + jnp.exp(-x))


def kernel(evidence_logits, t1_indices, t1_weights, t2_indices, t2_weights,
           t3_indices, t3_weights):
    B, N = evidence_logits.shape
    R = t1_weights.shape[0]

    SPC = -(-N // (TPR * U))
    SL = SPC * U
    NPAD = TPR * SL

    CPT = -(-R // (TPR * KCH))
    CPT += CPT % 2
    RPAD = TPR * CPT * KCH
    TROWS = CPT * CROWS

    ev = jnp.pad(evidence_logits, ((0, 0), (0, NPAD - N)))
    (a1, c1), w1 = _pad_cols(t1_indices, t1_weights, RPAD)
    (a2, b2, c2), w2 = _pad_cols(t2_indices, t2_weights, RPAD)
    (a3, b3, c3), w3 = _pad_cols(t3_indices, t3_weights, RPAD)

    mesh = plsc.VectorSubcoreMesh(core_axis_name="c", subcore_axis_name="s",
                                  num_cores=NCORE, num_subcores=NSUB)

    cbuf_i = pltpu.VMEM((CROWS, 128), jnp.int32)
    cbuf_f = pltpu.VMEM((CROWS, 128), jnp.float32)

    @functools.partial(
        pl.kernel,
        out_type=(jax.ShapeDtypeStruct((B, NPAD), jnp.float32),
                  jax.ShapeDtypeStruct((B, NPAD), jnp.float32)),
        mesh=mesh,
        compiler_params=pltpu.CompilerParams(needs_layout_passes=False,
                                             use_tc_tiling_on_sc=False),
        scratch_types=[
            pltpu.VMEM((NPAD,), jnp.float32),
            [cbuf_i] * 3 + [cbuf_f] * 4,
            [cbuf_i] * 3 + [cbuf_f] * 4,
            [cbuf_i] * 3,
            [cbuf_i] * 3,
            pltpu.VMEM((U,), jnp.float32),
            pltpu.VMEM((U,), jnp.float32),
            pltpu.VMEM((U,), jnp.float32),
            pltpu.VMEM_SHARED((2, NPAD), jnp.float32),
            pltpu.SemaphoreType.DMA,
            pltpu.SemaphoreType.DMA,
            pltpu.SemaphoreType.DMA,
            pltpu.SemaphoreType.DMA,
            pltpu.SemaphoreType.DMA,
            pltpu.SemaphoreType.DMA,
        ],
    )
    def cavi(ev_h, a1_h, c1_h, w1_h, a2_h, b2_h, c2_h, w2_h,
             a3_h, b3_h, c3_h, w3_h, q_h, lg_h,
             q_loc, bufs0, bufs1, sidx0, sidx1, lg_u, dl_u, ev_u,
             dl_sh, sem_in0, sem_in1, sem_sc, sem_u1, sem_u2, sem_u3):
        sem_in = (sem_in0, sem_in1)
        cid = lax.axis_index("c")
        sid = lax.axis_index("s")
        rl = sid // TPR
        b = 2 * cid + rl
        j = sid % TPR

        bufs = (bufs0, bufs1)
        sidx = (sidx0, sidx1)

        def update_slice(t):
            for u in range(SPC):
                off = j * SL + u * U
                cps = [
                    pltpu.async_copy(dl_sh.at[rl, pl.ds(off, U)], dl_u,
                                     sem_u1),
                    pltpu.async_copy(ev_h.at[b, pl.ds(off, U)], ev_u, sem_u2),
                    pltpu.async_copy(lg_h.at[b, pl.ds(off, U)], lg_u, sem_u3),
                ]
                for cp in cps:
                    cp.wait()

                def upd(v, _):
                    sl = pl.ds(v * LANES, LANES)
                    nl = ((1.0 - DAMPING) * lg_u[sl]
                          + DAMPING * (ev_u[sl] + dl_u[sl]))
                    lg_u[sl] = nl
                    dl_u[sl] = _sigmoid16(nl)
                    ev_u[sl] = jnp.zeros((LANES,), jnp.float32)
                    return _
                lax.fori_loop(0, U // LANES, upd, None)

                cps = [
                    pltpu.async_copy(lg_u, lg_h.at[b, pl.ds(off, U)], sem_u1),
                    pltpu.async_copy(dl_u, q_h.at[b, pl.ds(off, U)], sem_u2),
                    pltpu.async_copy(ev_u, dl_sh.at[rl, pl.ds(off, U)],
                                     sem_u3),
                ]
                for cp in cps:
                    cp.wait()

        def run_template(nops, idx_hs, w_h, compute):
            hs = list(idx_hs) + [w_h]

            def fire_input(ci, p):
                for h, buf in zip(hs, bufs[p][:nops] + [bufs[p][3]]):
                    row0 = j * TROWS + ci * CROWS
                    pltpu.async_copy(h.at[pl.ds(row0, CROWS)], buf, sem_in[p])

            def wait_input(ci, p):
                for h, buf in zip(hs, bufs[p][:nops] + [bufs[p][3]]):
                    row0 = j * TROWS + ci * CROWS
                    pltpu.make_async_copy(h.at[pl.ds(row0, CROWS)], buf,
                                          sem_in[p]).wait()

            def fire_scatter(p):
                for r in range(CROWS):
                    for mbuf, sbuf in zip(bufs[p][4:4 + nops], sidx[p]):
                        pltpu.async_copy(
                            mbuf.at[r], dl_sh.at[rl].at[sbuf.at[r]],
                            sem_sc, add=True)

            def wait_scatter(p):
                for mbuf in bufs[p][4:4 + nops]:
                    pltpu.make_async_copy(
                        w_h.at[pl.ds(0, CROWS)], mbuf, sem_sc).wait()

            def compute_chunk(p):
                ibs = bufs[p][:nops]
                wb = bufs[p][3]
                mbs = bufs[p][4:4 + nops]
                sbs = sidx[p]

                @plsc.parallel_loop(0, CROWS, 1)
                def grp(r):
                    for l in range(128 // LANES):
                        sl = pl.ds(l * LANES, LANES)
                        ivs = [buf[r, sl] for buf in ibs]
                        qs = [plsc.load_gather(q_loc, [iv]) for iv in ivs]
                        msgs = compute(wb[r, sl], *qs)
                        for mbuf, m in zip(mbs, msgs):
                            mbuf[r, sl] = m
                        for sbuf, iv in zip(sbs, ivs):
                            sbuf[r, sl] = iv

            def sub_body(ci, p, first=False):
                fire_input(lax.rem(ci + 1, CPT), 1 - p)
                wait_input(ci, p)
                compute_chunk(p)
                if not first:
                    wait_scatter(1 - p)
                fire_scatter(p)

            fire_input(0, 0)
            sub_body(0, 0, first=True)
            sub_body(1, 1)

            def pair(g, _):
                sub_body(2 * g, 0)
                sub_body(2 * g + 1, 1)
                return _
            lax.fori_loop(1, CPT // 2, pair, None)
            wait_scatter(1)
            wait_input(0, 0)

        def t1_msgs(w, qa, qc):
            t = qc - 1.0
            return (w * t, w * qa)

        def t2_msgs(w, qa, qb, qc):
            t = qc - 1.0
            wqa = w * qa
            return (w * qb * t, wqa * t, wqa * qb)

        def t3_msgs(w, qa, qb, qc):
            t = qc - 1.0
            u1 = 1.0 - qb
            wqa = w * qa
            return (w * u1 * t, wqa * (1.0 - qc), wqa * u1)

        for u in range(SPC):
            off = j * SL + u * U
            pltpu.sync_copy(ev_h.at[b, pl.ds(off, U)], ev_u)
            pltpu.sync_copy(ev_u, lg_h.at[b, pl.ds(off, U)])

            def sig(v, _):
                sl = pl.ds(v * LANES, LANES)
                dl_u[sl] = _sigmoid16(ev_u[sl])
                ev_u[sl] = jnp.zeros((LANES,), jnp.float32)
                return _
            lax.fori_loop(0, U // LANES, sig, None)
            pltpu.sync_copy(dl_u, q_h.at[b, pl.ds(off, U)])
            pltpu.sync_copy(ev_u, dl_sh.at[rl, pl.ds(off, U)])

        plsc.subcore_barrier()
        pltpu.sync_copy(q_h.at[b], q_loc)

        def iteration(t, _):
            run_template(2, [a1_h, c1_h], w1_h, t1_msgs)
            run_template(3, [a2_h, b2_h, c2_h], w2_h, t2_msgs)
            run_template(3, [a3_h, b3_h, c3_h], w3_h, t3_msgs)
            plsc.subcore_barrier()
            update_slice(t)
            plsc.subcore_barrier()

            @pl.when(t < MAX_ITERS - 1)
            def _():
                pltpu.sync_copy(q_h.at[b], q_loc)
            return _

        lax.fori_loop(0, MAX_ITERS, iteration, None)

    out, _lg = cavi(ev, a1, c1, w1, a2, b2, c2, w2, a3, b3, c3, w3)
    return out[:, :N]


# speedup vs baseline: 60.7031x; 1.0007x over previous
import functools

import jax
import jax.numpy as jnp
from jax import lax
from jax.experimental import pallas as pl
from jax.experimental.pallas import tpu as pltpu
from jax.experimental.pallas import tpu_sc as plsc

MAX_ITERS = 5
DAMPING = 0.5

NCORE = 2
NSUB = 16
LANES = 16
TPR = 8

KCH = 640
CROWS = KCH // 128
U = 1568


def _pad_cols(idx, w, r_pad):
    r = w.shape[0]
    cols = [jnp.pad(idx[:, k], (0, r_pad - r)).reshape(-1, 128)
            for k in range(idx.shape[1])]
    wp = jnp.pad(w, (0, r_pad - r)).reshape(-1, 128)
    return cols, wp


def _sigmoid16(x):
    return 1.0 / (1.0 + jnp.exp(-x))


def kernel(evidence_logits, t1_indices, t1_weights, t2_indices, t2_weights,
           t3_indices, t3_weights):
    B, N = evidence_logits.shape
    R = t1_weights.shape[0]

    SPC = -(-N // (TPR * U))
    SL = SPC * U
    NPAD = TPR * SL

    CPT = -(-R // (TPR * KCH))
    CPT += CPT % 2
    RPAD = TPR * CPT * KCH
    TROWS = CPT * CROWS

    ev = jnp.pad(evidence_logits, ((0, 0), (0, NPAD - N)))
    (a1, c1), w1 = _pad_cols(t1_indices, t1_weights, RPAD)
    (a2, b2, c2), w2 = _pad_cols(t2_indices, t2_weights, RPAD)
    (a3, b3, c3), w3 = _pad_cols(t3_indices, t3_weights, RPAD)

    mesh = plsc.VectorSubcoreMesh(core_axis_name="c", subcore_axis_name="s",
                                  num_cores=NCORE, num_subcores=NSUB)

    cbuf_i = pltpu.VMEM((CROWS, 128), jnp.int32)
    cbuf_f = pltpu.VMEM((CROWS, 128), jnp.float32)

    @functools.partial(
        pl.kernel,
        out_type=(jax.ShapeDtypeStruct((B, NPAD), jnp.float32),
                  jax.ShapeDtypeStruct((B, NPAD), jnp.float32)),
        mesh=mesh,
        compiler_params=pltpu.CompilerParams(needs_layout_passes=False,
                                             use_tc_tiling_on_sc=False),
        scratch_types=[
            pltpu.VMEM((NPAD,), jnp.float32),
            [cbuf_i] * 3 + [cbuf_f] * 4,
            [cbuf_i] * 3 + [cbuf_f] * 4,
            [cbuf_i] * 3,
            [cbuf_i] * 3,
            pltpu.VMEM((U,), jnp.float32),
            pltpu.VMEM((U,), jnp.float32),
            pltpu.VMEM((U,), jnp.float32),
            pltpu.VMEM_SHARED((2, NPAD), jnp.float32),
            pltpu.SemaphoreType.DMA,
            pltpu.SemaphoreType.DMA,
            pltpu.SemaphoreType.DMA,
            pltpu.SemaphoreType.DMA,
            pltpu.SemaphoreType.DMA,
            pltpu.SemaphoreType.DMA,
        ],
    )
    def cavi(ev_h, a1_h, c1_h, w1_h, a2_h, b2_h, c2_h, w2_h,
             a3_h, b3_h, c3_h, w3_h, q_h, lg_h,
             q_loc, bufs0, bufs1, sidx0, sidx1, lg_u, dl_u, ev_u,
             dl_sh, sem_in0, sem_in1, sem_sc, sem_u1, sem_u2, sem_u3):
        sem_in = (sem_in0, sem_in1)
        cid = lax.axis_index("c")
        sid = lax.axis_index("s")
        rl = sid // TPR
        b = 2 * cid + rl
        j = sid % TPR

        bufs = (bufs0, bufs1)
        sidx = (sidx0, sidx1)

        def update_slice(t):
            for u in range(SPC):
                off = j * SL + u * U
                cps = [
                    pltpu.async_copy(dl_sh.at[rl, pl.ds(off, U)], dl_u,
                                     sem_u1),
                    pltpu.async_copy(ev_h.at[b, pl.ds(off, U)], ev_u, sem_u2),
                    pltpu.async_copy(lg_h.at[b, pl.ds(off, U)], lg_u, sem_u3),
                ]
                for cp in cps:
                    cp.wait()

                def upd(v, _):
                    sl = pl.ds(v * LANES, LANES)
                    nl = ((1.0 - DAMPING) * lg_u[sl]
                          + DAMPING * (ev_u[sl] + dl_u[sl]))
                    lg_u[sl] = nl
                    dl_u[sl] = _sigmoid16(nl)
                    ev_u[sl] = jnp.zeros((LANES,), jnp.float32)
                    return _
                lax.fori_loop(0, U // LANES, upd, None)

                cps = [
                    pltpu.async_copy(lg_u, lg_h.at[b, pl.ds(off, U)], sem_u1),
                    pltpu.async_copy(dl_u, q_h.at[b, pl.ds(off, U)], sem_u2),
                    pltpu.async_copy(ev_u, dl_sh.at[rl, pl.ds(off, U)],
                                     sem_u3),
                ]
                for cp in cps:
                    cp.wait()

        def run_template(nops, idx_hs, w_h, compute):
            hs = list(idx_hs) + [w_h]

            def fire_input(ci, p):
                for h, buf in zip(hs, bufs[p][:nops] + [bufs[p][3]]):
                    row0 = j * TROWS + ci * CROWS
                    pltpu.async_copy(h.at[pl.ds(row0, CROWS)], buf, sem_in[p])

            def wait_input(ci, p):
                for h, buf in zip(hs, bufs[p][:nops] + [bufs[p][3]]):
                    row0 = j * TROWS + ci * CROWS
                    pltpu.make_async_copy(h.at[pl.ds(row0, CROWS)], buf,
                                          sem_in[p]).wait()

            def fire_scatter(p):
                for r in range(CROWS):
                    for mbuf, sbuf in zip(bufs[p][4:4 + nops], sidx[p]):
                        pltpu.async_copy(
                            mbuf.at[r], dl_sh.at[rl].at[sbuf.at[r]],
                            sem_sc, add=True)

            def wait_scatter(p):
                for mbuf in bufs[p][4:4 + nops]:
                    pltpu.make_async_copy(
                        w_h.at[pl.ds(0, CROWS)], mbuf, sem_sc).wait()

            def compute_chunk(p):
                ibs = bufs[p][:nops]
                wb = bufs[p][3]
                mbs = bufs[p][4:4 + nops]
                sbs = sidx[p]

                @plsc.parallel_loop(0, CROWS, 1, unroll=CROWS)
                def grp(r):
                    for l in range(128 // LANES):
                        sl = pl.ds(l * LANES, LANES)
                        ivs = [buf[r, sl] for buf in ibs]
                        qs = [plsc.load_gather(q_loc, [iv]) for iv in ivs]
                        msgs = compute(wb[r, sl], *qs)
                        for mbuf, m in zip(mbs, msgs):
                            mbuf[r, sl] = m
                        for sbuf, iv in zip(sbs, ivs):
                            sbuf[r, sl] = iv

            def sub_body(ci, p, first=False):
                fire_input(lax.rem(ci + 1, CPT), 1 - p)
                wait_input(ci, p)
                compute_chunk(p)
                if not first:
                    wait_scatter(1 - p)
                fire_scatter(p)

            fire_input(0, 0)
            sub_body(0, 0, first=True)
            sub_body(1, 1)

            def pair(g, _):
                sub_body(2 * g, 0)
                sub_body(2 * g + 1, 1)
                return _
            lax.fori_loop(1, CPT // 2, pair, None)
            wait_scatter(1)
            wait_input(0, 0)

        def t1_msgs(w, qa, qc):
            t = qc - 1.0
            return (w * t, w * qa)

        def t2_msgs(w, qa, qb, qc):
            t = qc - 1.0
            wqa = w * qa
            return (w * qb * t, wqa * t, wqa * qb)

        def t3_msgs(w, qa, qb, qc):
            t = qc - 1.0
            u1 = 1.0 - qb
            wqa = w * qa
            return (w * u1 * t, wqa * (1.0 - qc), wqa * u1)

        for u in range(SPC):
            off = j * SL + u * U
            pltpu.sync_copy(ev_h.at[b, pl.ds(off, U)], ev_u)
            pltpu.sync_copy(ev_u, lg_h.at[b, pl.ds(off, U)])

            def sig(v, _):
                sl = pl.ds(v * LANES, LANES)
                dl_u[sl] = _sigmoid16(ev_u[sl])
                ev_u[sl] = jnp.zeros((LANES,), jnp.float32)
                return _
            lax.fori_loop(0, U // LANES, sig, None)
            pltpu.sync_copy(dl_u, q_h.at[b, pl.ds(off, U)])
            pltpu.sync_copy(ev_u, dl_sh.at[rl, pl.ds(off, U)])

        plsc.subcore_barrier()
        pltpu.sync_copy(q_h.at[b], q_loc)

        def iteration(t, _):
            run_template(2, [a1_h, c1_h], w1_h, t1_msgs)
            run_template(3, [a2_h, b2_h, c2_h], w2_h, t2_msgs)
            run_template(3, [a3_h, b3_h, c3_h], w3_h, t3_msgs)
            plsc.subcore_barrier()
            update_slice(t)
            plsc.subcore_barrier()

            @pl.when(t < MAX_ITERS - 1)
            def _():
                pltpu.sync_copy(q_h.at[b], q_loc)
            return _

        lax.fori_loop(0, MAX_ITERS, iteration, None)

    out, _lg = cavi(ev, a1, c1, w1, a2, b2, c2, w2, a3, b3, c3, w3)
    return out[:, :N]


# speedup vs baseline: 61.4495x; 1.0123x over previous
import functools

import jax
import jax.numpy as jnp
from jax import lax
from jax.experimental import pallas as pl
from jax.experimental.pallas import tpu as pltpu
from jax.experimental.pallas import tpu_sc as plsc

MAX_ITERS = 5
DAMPING = 0.5

NCORE = 2
NSUB = 16
LANES = 16
TPR = 8

KCH = 640
CROWS = KCH // 128
U = 1568


def _pad_cols(idx, w, r_pad):
    r = w.shape[0]
    cols = [jnp.pad(idx[:, k], (0, r_pad - r)).reshape(-1, 128)
            for k in range(idx.shape[1])]
    wp = jnp.pad(w, (0, r_pad - r)).reshape(-1, 128)
    return cols, wp


def _sigmoid16(x):
    return 1.0 / (1.0 + jnp.exp(-x))


def kernel(evidence_logits, t1_indices, t1_weights, t2_indices, t2_weights,
           t3_indices, t3_weights):
    B, N = evidence_logits.shape
    R = t1_weights.shape[0]

    SPC = -(-N // (TPR * U))
    SL = SPC * U
    NPAD = TPR * SL

    CPT = -(-R // (TPR * KCH))
    CPT += CPT % 2
    RPAD = TPR * CPT * KCH
    TROWS = CPT * CROWS

    ev = jnp.pad(evidence_logits, ((0, 0), (0, NPAD - N)))
    (a1, c1), w1 = _pad_cols(t1_indices, t1_weights, RPAD)
    (a2, b2, c2), w2 = _pad_cols(t2_indices, t2_weights, RPAD)
    (a3, b3, c3), w3 = _pad_cols(t3_indices, t3_weights, RPAD)

    mesh = plsc.VectorSubcoreMesh(core_axis_name="c", subcore_axis_name="s",
                                  num_cores=NCORE, num_subcores=NSUB)

    cbuf_i = pltpu.VMEM((CROWS, 128), jnp.int32)
    cbuf_f = pltpu.VMEM((CROWS, 128), jnp.float32)

    @functools.partial(
        pl.kernel,
        out_type=(jax.ShapeDtypeStruct((B, NPAD), jnp.float32),
                  jax.ShapeDtypeStruct((B, NPAD), jnp.float32)),
        mesh=mesh,
        compiler_params=pltpu.CompilerParams(needs_layout_passes=False,
                                             use_tc_tiling_on_sc=False),
        scratch_types=[
            pltpu.VMEM((NPAD,), jnp.float32),
            [cbuf_i] * 3 + [cbuf_f] * 4,
            [cbuf_i] * 3 + [cbuf_f] * 4,
            [cbuf_i] * 3,
            [cbuf_i] * 3,
            pltpu.VMEM((U,), jnp.float32),
            pltpu.VMEM((U,), jnp.float32),
            pltpu.VMEM((U,), jnp.float32),
            pltpu.VMEM_SHARED((2, NPAD), jnp.float32),
            pltpu.SemaphoreType.DMA,
            pltpu.SemaphoreType.DMA,
            pltpu.SemaphoreType.DMA,
            pltpu.SemaphoreType.DMA,
            pltpu.SemaphoreType.DMA,
            pltpu.SemaphoreType.DMA,
            pltpu.SemaphoreType.DMA,
        ],
    )
    def cavi(ev_h, a1_h, c1_h, w1_h, a2_h, b2_h, c2_h, w2_h,
             a3_h, b3_h, c3_h, w3_h, q_h, lg_h,
             q_loc, bufs0, bufs1, sidx0, sidx1, lg_u, dl_u, ev_u,
             dl_sh, sem_in0, sem_in1, sem_sc0, sem_sc1,
             sem_u1, sem_u2, sem_u3):
        sem_in = (sem_in0, sem_in1)
        sem_sc = (sem_sc0, sem_sc1)
        cid = lax.axis_index("c")
        sid = lax.axis_index("s")
        rl = sid // TPR
        b = 2 * cid + rl
        j = sid % TPR

        bufs = (bufs0, bufs1)
        sidx = (sidx0, sidx1)

        def update_slice(t):
            for u in range(SPC):
                off = j * SL + u * U
                cps = [
                    pltpu.async_copy(dl_sh.at[rl, pl.ds(off, U)], dl_u,
                                     sem_u1),
                    pltpu.async_copy(ev_h.at[b, pl.ds(off, U)], ev_u, sem_u2),
                    pltpu.async_copy(lg_h.at[b, pl.ds(off, U)], lg_u, sem_u3),
                ]
                for cp in cps:
                    cp.wait()

                def upd(v, _):
                    sl = pl.ds(v * LANES, LANES)
                    nl = ((1.0 - DAMPING) * lg_u[sl]
                          + DAMPING * (ev_u[sl] + dl_u[sl]))
                    lg_u[sl] = nl
                    dl_u[sl] = _sigmoid16(nl)
                    ev_u[sl] = jnp.zeros((LANES,), jnp.float32)
                    return _
                lax.fori_loop(0, U // LANES, upd, None)

                cps = [
                    pltpu.async_copy(lg_u, lg_h.at[b, pl.ds(off, U)], sem_u1),
                    pltpu.async_copy(dl_u, q_h.at[b, pl.ds(off, U)], sem_u2),
                    pltpu.async_copy(ev_u, dl_sh.at[rl, pl.ds(off, U)],
                                     sem_u3),
                ]
                for cp in cps:
                    cp.wait()

        def run_template(nops, idx_hs, w_h, compute):
            hs = list(idx_hs) + [w_h]

            def fire_input(ci, p):
                for h, buf in zip(hs, bufs[p][:nops] + [bufs[p][3]]):
                    row0 = j * TROWS + ci * CROWS
                    pltpu.async_copy(h.at[pl.ds(row0, CROWS)], buf, sem_in[p])

            def wait_input(ci, p):
                for h, buf in zip(hs, bufs[p][:nops] + [bufs[p][3]]):
                    row0 = j * TROWS + ci * CROWS
                    pltpu.make_async_copy(h.at[pl.ds(row0, CROWS)], buf,
                                          sem_in[p]).wait()

            def fire_scatter(p):
                for r in range(CROWS):
                    for mbuf, sbuf in zip(bufs[p][4:4 + nops], sidx[p]):
                        pltpu.async_copy(
                            mbuf.at[r], dl_sh.at[rl].at[sbuf.at[r]],
                            sem_sc[p], add=True)

            def wait_scatter(p):
                for mbuf in bufs[p][4:4 + nops]:
                    pltpu.make_async_copy(
                        w_h.at[pl.ds(0, CROWS)], mbuf, sem_sc[p]).wait()

            def compute_chunk(p):
                ibs = bufs[p][:nops]
                wb = bufs[p][3]
                mbs = bufs[p][4:4 + nops]
                sbs = sidx[p]

                @plsc.parallel_loop(0, CROWS, 1, unroll=CROWS)
                def grp(r):
                    for l in range(128 // LANES):
                        sl = pl.ds(l * LANES, LANES)
                        ivs = [buf[r, sl] for buf in ibs]
                        qs = [plsc.load_gather(q_loc, [iv]) for iv in ivs]
                        msgs = compute(wb[r, sl], *qs)
                        for mbuf, m in zip(mbs, msgs):
                            mbuf[r, sl] = m
                        for sbuf, iv in zip(sbs, ivs):
                            sbuf[r, sl] = iv

            def sub_body(ci, p, first=False):
                fire_input(lax.rem(ci + 1, CPT), 1 - p)
                wait_input(ci, p)
                compute_chunk(p)
                fire_scatter(p)
                if not first:
                    wait_scatter(1 - p)

            fire_input(0, 0)
            sub_body(0, 0, first=True)
            sub_body(1, 1)

            def pair(g, _):
                sub_body(2 * g, 0)
                sub_body(2 * g + 1, 1)
                return _
            lax.fori_loop(1, CPT // 2, pair, None)
            wait_scatter(1)
            wait_input(0, 0)

        def t1_msgs(w, qa, qc):
            t = qc - 1.0
            return (w * t, w * qa)

        def t2_msgs(w, qa, qb, qc):
            t = qc - 1.0
            wqa = w * qa
            return (w * qb * t, wqa * t, wqa * qb)

        def t3_msgs(w, qa, qb, qc):
            t = qc - 1.0
            u1 = 1.0 - qb
            wqa = w * qa
            return (w * u1 * t, wqa * (1.0 - qc), wqa * u1)

        for u in range(SPC):
            off = j * SL + u * U
            pltpu.sync_copy(ev_h.at[b, pl.ds(off, U)], ev_u)
            pltpu.sync_copy(ev_u, lg_h.at[b, pl.ds(off, U)])

            def sig(v, _):
                sl = pl.ds(v * LANES, LANES)
                dl_u[sl] = _sigmoid16(ev_u[sl])
                ev_u[sl] = jnp.zeros((LANES,), jnp.float32)
                return _
            lax.fori_loop(0, U // LANES, sig, None)
            pltpu.sync_copy(dl_u, q_h.at[b, pl.ds(off, U)])
            pltpu.sync_copy(ev_u, dl_sh.at[rl, pl.ds(off, U)])

        plsc.subcore_barrier()
        pltpu.sync_copy(q_h.at[b], q_loc)

        def iteration(t, _):
            run_template(2, [a1_h, c1_h], w1_h, t1_msgs)
            run_template(3, [a2_h, b2_h, c2_h], w2_h, t2_msgs)
            run_template(3, [a3_h, b3_h, c3_h], w3_h, t3_msgs)
            plsc.subcore_barrier()
            update_slice(t)
            plsc.subcore_barrier()

            @pl.when(t < MAX_ITERS - 1)
            def _():
                pltpu.sync_copy(q_h.at[b], q_loc)
            return _

        lax.fori_loop(0, MAX_ITERS, iteration, None)

    out, _lg = cavi(ev, a1, c1, w1, a2, b2, c2, w2, a3, b3, c3, w3)
    return out[:, :N]
